# Initial kernel scaffold; baseline (speedup 1.0000x reference)
import jax
import jax.numpy as jnp
from jax.experimental import pallas as pl


def kernel(features, target):
    raise NotImplementedError("write your pallas kernel here")


# speedup vs baseline: 1.0553x; 1.0553x over previous
import functools

import jax
import jax.numpy as jnp
from jax.experimental import pallas as pl
from jax.experimental.pallas import tpu as pltpu

_MARGIN = 1.0
_BLK = 512


def _loss_block(fi_ref, fj_ref, tcol_ref, trow_ref, out_ref, *, ni):
    jj = pl.program_id(1)

    @pl.when(jj == 0)
    def _init():
        out_ref[...] = jnp.zeros_like(out_ref)

    fi = fi_ref[...]
    fj = fj_ref[...]
    bi = fi.shape[0]
    bj = fj.shape[0]

    g = jax.lax.dot_general(fi, fj, (((1,), (1,)), ((), ())),
                            preferred_element_type=jnp.float32)
    sqi = jnp.sum(fi * fi, axis=1, keepdims=True)
    ones_row = jnp.ones((1, fj.shape[1]), jnp.float32)
    sqj = jax.lax.dot_general(ones_row, fj * fj, (((1,), (1,)), ((), ())),
                              precision=jax.lax.Precision.HIGHEST,
                              preferred_element_type=jnp.float32)
    d = jnp.maximum(sqi + sqj - 2.0 * g, 0.0)

    ti = tcol_ref[0, :, 0:1]
    tj = trow_ref[0]
    same = ti == tj

    loc_eye = (jax.lax.broadcasted_iota(jnp.int32, (bi, bj), 0)
               == jax.lax.broadcasted_iota(jnp.int32, (bi, bj), 1))
    eye = jnp.logical_and(jj == 0, loc_eye)

    pos = jnp.where(same & ~eye, d, 0.0)
    tmp = _MARGIN - jnp.sqrt(d)
    neg = jnp.where((~same) & (tmp > 0.0), tmp, 0.0)

    w = jnp.where((jj == 0) | (jj * 2 == ni), 1.0, 2.0)
    out_ref[...] += w * jnp.sum(pos + neg * neg)


def kernel(features, target):
    n, c = features.shape
    blk = _BLK if n % _BLK == 0 else n
    ni = n // blk
    njj = ni // 2 + 1

    t32 = target.astype(jnp.int32)
    trow = t32.reshape(ni, 1, blk)
    tcol = jnp.broadcast_to(t32[:, None], (n, 128)).reshape(ni, blk, 128)

    grid = (ni, njj)
    partials = pl.pallas_call(
        functools.partial(_loss_block, ni=ni),
        grid=grid,
        in_specs=[
            pl.BlockSpec((blk, c), lambda i, jj: (i, 0)),
            pl.BlockSpec((blk, c), lambda i, jj: ((i + jj) % ni, 0)),
            pl.BlockSpec((1, blk, 128), lambda i, jj: (i, 0, 0)),
            pl.BlockSpec((1, 1, blk), lambda i, jj: ((i + jj) % ni, 0, 0)),
        ],
        out_specs=pl.BlockSpec((1, 1, 128), lambda i, jj: (i, 0, 0)),
        out_shape=jax.ShapeDtypeStruct((ni, 1, 128), jnp.float32),
        compiler_params=pltpu.CompilerParams(
            dimension_semantics=("parallel", "arbitrary"),
        ),
    )(features, features, tcol, trow)

    t = n * (n - 1)
    return jnp.sum(partials[:, 0, 0]) / (2.0 * t)


# speedup vs baseline: 1.1385x; 1.0789x over previous
import functools

import jax
import jax.numpy as jnp
from jax.experimental import pallas as pl
from jax.experimental.pallas import tpu as pltpu

_MARGIN = 1.0
_BLK = 512


def _loss_block(fi_ref, fj_ref, tcol_ref, trow_ref, out_ref, *, ni):
    jj = pl.program_id(1)

    @pl.when(jj == 0)
    def _init():
        out_ref[...] = jnp.zeros_like(out_ref)

    fi = fi_ref[...]
    fj = fj_ref[...]
    bi = fi.shape[0]
    bj = fj.shape[0]

    g = jax.lax.dot_general(fi, fj, (((1,), (1,)), ((), ())),
                            preferred_element_type=jnp.float32)
    fi32 = fi.astype(jnp.float32)
    fj32 = fj.astype(jnp.float32)
    sqi = jnp.sum(fi32 * fi32, axis=1, keepdims=True)
    ones_row = jnp.ones((1, fj.shape[1]), jnp.float32)
    sqj = jax.lax.dot_general(ones_row, fj32 * fj32, (((1,), (1,)), ((), ())),
                              precision=jax.lax.Precision.HIGHEST,
                              preferred_element_type=jnp.float32)
    d = jnp.maximum((sqi + sqj) - 2.0 * g, 0.0)

    same = tcol_ref[0] == trow_ref[0]

    s = jnp.sqrt(d)
    r = jnp.maximum(_MARGIN - s, 0.0)
    val = jnp.where(same, d, r * r)

    w = jnp.where((jj == 0) | (jj * 2 == ni), 1.0, 2.0)
    out_ref[...] += w * jnp.sum(val)

    @pl.when(jj == 0)
    def _diag_correction():
        loc_eye = (jax.lax.broadcasted_iota(jnp.int32, (bi, bj), 0)
                   == jax.lax.broadcasted_iota(jnp.int32, (bi, bj), 1))
        out_ref[...] -= jnp.sum(jnp.where(loc_eye, d, 0.0))


def kernel(features, target):
    n, c = features.shape
    blk = _BLK if n % _BLK == 0 else n
    ni = n // blk
    njj = ni // 2 + 1

    fb = features.astype(jnp.bfloat16)
    tf = target.astype(jnp.float32)
    trow = tf.reshape(ni, 1, blk)
    tcol = jnp.broadcast_to(tf[:, None], (n, blk)).reshape(ni, blk, blk)

    grid = (ni, njj)
    partials = pl.pallas_call(
        functools.partial(_loss_block, ni=ni),
        grid=grid,
        in_specs=[
            pl.BlockSpec((blk, c), lambda i, jj: (i, 0)),
            pl.BlockSpec((blk, c), lambda i, jj: ((i + jj) % ni, 0)),
            pl.BlockSpec((1, blk, blk), lambda i, jj: (i, 0, 0)),
            pl.BlockSpec((1, 1, blk), lambda i, jj: ((i + jj) % ni, 0, 0)),
        ],
        out_specs=pl.BlockSpec((1, 1, 128), lambda i, jj: (i, 0, 0)),
        out_shape=jax.ShapeDtypeStruct((ni, 1, 128), jnp.float32),
        compiler_params=pltpu.CompilerParams(
            dimension_semantics=("parallel", "arbitrary"),
        ),
    )(fb, fb, tcol, trow)

    t = n * (n - 1)
    return jnp.sum(partials[:, 0, 0]) / (2.0 * t)


# speedup vs baseline: 1.7086x; 1.5007x over previous
import functools

import jax
import jax.numpy as jnp
from jax.experimental import pallas as pl
from jax.experimental.pallas import tpu as pltpu

_MARGIN = 1.0
_BLK = 512


def _norms_block(f_ref, row_ref, col_ref):
    f32 = f_ref[...].astype(jnp.float32)
    sq2 = f32 * f32
    col = jnp.sum(sq2, axis=1, keepdims=True)
    ones_row = jnp.ones((1, f32.shape[1]), jnp.float32)
    row = jax.lax.dot_general(ones_row, sq2, (((1,), (1,)), ((), ())),
                              precision=jax.lax.Precision.HIGHEST,
                              preferred_element_type=jnp.float32)
    row_ref[0] = row
    col_ref[0] = jnp.broadcast_to(col, col_ref.shape[1:])


def _loss_block(fi_ref, fj_ref, sqc_ref, sqr_ref, tcol_ref, trow_ref,
                out_ref, *, ni):
    jj = pl.program_id(1)

    @pl.when(jj == 0)
    def _init():
        out_ref[...] = jnp.zeros_like(out_ref)

    fi = fi_ref[...]
    fj = fj_ref[...]
    bi = fi.shape[0]
    bj = fj.shape[0]

    g = jax.lax.dot_general(fi, fj, (((1,), (1,)), ((), ())),
                            preferred_element_type=jnp.float32)
    sqc = sqc_ref[0, :, 0:1]
    sqr = sqr_ref[0]
    d = jnp.maximum((sqc + sqr) - 2.0 * g, 0.0)

    same = tcol_ref[0] == trow_ref[0]

    s = jnp.sqrt(d)
    r = jnp.maximum(_MARGIN - s, 0.0)
    val = jnp.where(same, d, r * r)

    w = jnp.where((jj == 0) | (jj * 2 == ni), 1.0, 2.0)
    out_ref[...] += w * jnp.sum(val)

    @pl.when(jj == 0)
    def _diag_correction():
        loc_eye = (jax.lax.broadcasted_iota(jnp.int32, (bi, bj), 0)
                   == jax.lax.broadcasted_iota(jnp.int32, (bi, bj), 1))
        out_ref[...] -= jnp.sum(jnp.where(loc_eye, d, 0.0))


def kernel(features, target):
    n, c = features.shape
    blk = _BLK if n % _BLK == 0 else n
    ni = n // blk
    njj = ni // 2 + 1

    fb = features.astype(jnp.bfloat16)
    tf = target.astype(jnp.float32)
    trow = tf.reshape(ni, 1, blk)
    tcol = jnp.broadcast_to(tf[:, None], (n, blk)).reshape(ni, blk, blk)

    sq_row, sq_col = pl.pallas_call(
        _norms_block,
        grid=(ni,),
        in_specs=[pl.BlockSpec((blk, c), lambda i: (i, 0))],
        out_specs=[
            pl.BlockSpec((1, 1, blk), lambda i: (i, 0, 0)),
            pl.BlockSpec((1, blk, 128), lambda i: (i, 0, 0)),
        ],
        out_shape=[
            jax.ShapeDtypeStruct((ni, 1, blk), jnp.float32),
            jax.ShapeDtypeStruct((ni, blk, 128), jnp.float32),
        ],
        compiler_params=pltpu.CompilerParams(
            dimension_semantics=("parallel",),
        ),
    )(fb)

    grid = (ni, njj)
    partials = pl.pallas_call(
        functools.partial(_loss_block, ni=ni),
        grid=grid,
        in_specs=[
            pl.BlockSpec((blk, c), lambda i, jj: (i, 0)),
            pl.BlockSpec((blk, c), lambda i, jj: ((i + jj) % ni, 0)),
            pl.BlockSpec((1, blk, 128), lambda i, jj: (i, 0, 0)),
            pl.BlockSpec((1, 1, blk), lambda i, jj: ((i + jj) % ni, 0, 0)),
            pl.BlockSpec((1, blk, blk), lambda i, jj: (i, 0, 0)),
            pl.BlockSpec((1, 1, blk), lambda i, jj: ((i + jj) % ni, 0, 0)),
        ],
        out_specs=pl.BlockSpec((1, 1, 128), lambda i, jj: (i, 0, 0)),
        out_shape=jax.ShapeDtypeStruct((ni, 1, 128), jnp.float32),
        compiler_params=pltpu.CompilerParams(
            dimension_semantics=("parallel", "arbitrary"),
        ),
    )(fb, fb, sq_col, sq_row, tcol, trow)

    t = n * (n - 1)
    return jnp.sum(partials[:, 0, 0]) / (2.0 * t)


# speedup vs baseline: 2.2124x; 1.2948x over previous
import functools

import jax
import jax.numpy as jnp
from jax.experimental import pallas as pl
from jax.experimental.pallas import tpu as pltpu

_MARGIN = 1.0
_BLK = 1024


def _norms_block(f_ref, row_ref, col_ref):
    f32 = f_ref[...].astype(jnp.float32)
    sq2 = f32 * f32
    col = jnp.sum(sq2, axis=1, keepdims=True)
    ones_row = jnp.ones((1, f32.shape[1]), jnp.float32)
    row = jax.lax.dot_general(ones_row, sq2, (((1,), (1,)), ((), ())),
                              precision=jax.lax.Precision.HIGHEST,
                              preferred_element_type=jnp.float32)
    row_ref[0] = row
    col_ref[0] = jnp.broadcast_to(col, col_ref.shape[1:])


def _loss_block(fi_ref, fj_ref, sqc_ref, sqr_ref, tcol_ref, trow_ref,
                out_ref, *, ni):
    jj = pl.program_id(1)

    @pl.when(jj == 0)
    def _init():
        out_ref[...] = jnp.zeros_like(out_ref)

    fi = fi_ref[...]
    fj = fj_ref[...]
    bi = fi.shape[0]
    bj = fj.shape[0]

    g = jax.lax.dot_general(fi, fj, (((1,), (1,)), ((), ())),
                            preferred_element_type=jnp.float32)
    sqc = sqc_ref[0, :, 0:1]
    sqr = sqr_ref[0]
    d = jnp.maximum((sqc + sqr) - 2.0 * g, 0.0)

    same = tcol_ref[0] == trow_ref[0]

    s = jnp.sqrt(d)
    r = jnp.maximum(_MARGIN - s, 0.0)
    val = jnp.where(same, d, r * r)

    w = jnp.where((jj == 0) | (jj * 2 == ni), 1.0, 2.0)
    out_ref[...] += w * jnp.sum(val)

    @pl.when(jj == 0)
    def _diag_correction():
        loc_eye = (jax.lax.broadcasted_iota(jnp.int32, (bi, bj), 0)
                   == jax.lax.broadcasted_iota(jnp.int32, (bi, bj), 1))
        out_ref[...] -= jnp.sum(jnp.where(loc_eye, d, 0.0))


def kernel(features, target):
    n, c = features.shape
    blk = _BLK if n % _BLK == 0 else n
    ni = n // blk
    njj = ni // 2 + 1

    fb = features.astype(jnp.bfloat16)
    tf = target.astype(jnp.float32)
    trow = tf.reshape(ni, 1, blk)
    tcol = jnp.broadcast_to(tf[:, None], (n, blk)).reshape(ni, blk, blk)

    sq_row, sq_col = pl.pallas_call(
        _norms_block,
        grid=(ni,),
        in_specs=[pl.BlockSpec((blk, c), lambda i: (i, 0))],
        out_specs=[
            pl.BlockSpec((1, 1, blk), lambda i: (i, 0, 0)),
            pl.BlockSpec((1, blk, 128), lambda i: (i, 0, 0)),
        ],
        out_shape=[
            jax.ShapeDtypeStruct((ni, 1, blk), jnp.float32),
            jax.ShapeDtypeStruct((ni, blk, 128), jnp.float32),
        ],
        compiler_params=pltpu.CompilerParams(
            dimension_semantics=("parallel",),
        ),
    )(fb)

    grid = (ni, njj)
    partials = pl.pallas_call(
        functools.partial(_loss_block, ni=ni),
        grid=grid,
        in_specs=[
            pl.BlockSpec((blk, c), lambda i, jj: (i, 0)),
            pl.BlockSpec((blk, c), lambda i, jj: ((i + jj) % ni, 0)),
            pl.BlockSpec((1, blk, 128), lambda i, jj: (i, 0, 0)),
            pl.BlockSpec((1, 1, blk), lambda i, jj: ((i + jj) % ni, 0, 0)),
            pl.BlockSpec((1, blk, blk), lambda i, jj: (i, 0, 0)),
            pl.BlockSpec((1, 1, blk), lambda i, jj: ((i + jj) % ni, 0, 0)),
        ],
        out_specs=pl.BlockSpec((1, 1, 128), lambda i, jj: (i, 0, 0)),
        out_shape=jax.ShapeDtypeStruct((ni, 1, 128), jnp.float32),
        compiler_params=pltpu.CompilerParams(
            dimension_semantics=("parallel", "arbitrary"),
        ),
    )(fb, fb, sq_col, sq_row, tcol, trow)

    t = n * (n - 1)
    return jnp.sum(partials[:, 0, 0]) / (2.0 * t)


# speedup vs baseline: 2.8713x; 1.2979x over previous
import functools

import jax
import jax.numpy as jnp
from jax.experimental import pallas as pl
from jax.experimental.pallas import tpu as pltpu

_MARGIN = 1.0
_BLK = 1024


def _norms_block(f_ref, row_ref, col_ref, fm2_ref):
    f = f_ref[...]
    fm2_ref[...] = f * jnp.bfloat16(-2.0)
    f32 = f.astype(jnp.float32)
    sq2 = f32 * f32
    col = jnp.sum(sq2, axis=1, keepdims=True)
    ones_row = jnp.ones((1, f32.shape[1]), jnp.float32)
    row = jax.lax.dot_general(ones_row, sq2, (((1,), (1,)), ((), ())),
                              precision=jax.lax.Precision.HIGHEST,
                              preferred_element_type=jnp.float32)
    row_ref[0] = row
    col_ref[0] = jnp.broadcast_to(col, col_ref.shape[1:])


def _loss_block(fi_ref, fj_ref, sqc_ref, sqr_ref, tcol_ref, trow_ref,
                out_ref, *, ni):
    jj = pl.program_id(1)

    @pl.when(jj == 0)
    def _init():
        out_ref[...] = jnp.zeros_like(out_ref)

    fi = fi_ref[...]
    bi = fi_ref.shape[0]
    bj = fj_ref.shape[0]
    half = bj // 2
    sqc = sqc_ref[0, :, 0:1]
    ti = tcol_ref[0, :, 0:1]

    def _half(lo):
        fj = fj_ref[pl.ds(lo, half), :]
        g2 = jax.lax.dot_general(fi, fj, (((1,), (1,)), ((), ())),
                                 preferred_element_type=jnp.float32)
        sqr = sqr_ref[0, :, pl.ds(lo, half)]
        d = jnp.maximum((sqc + sqr) + g2, 1e-20)
        same = ti == trow_ref[0, :, pl.ds(lo, half)]
        s = d * jax.lax.rsqrt(d)
        r = jnp.maximum(_MARGIN - s, 0.0)
        val = jnp.where(same, d, r * r)
        return d, jnp.sum(val)

    d_a, sum_a = _half(0)
    d_b, sum_b = _half(half)

    w = jnp.where((jj == 0) | (jj * 2 == ni), 1.0, 2.0)
    out_ref[...] += w * (sum_a + sum_b)

    @pl.when(jj == 0)
    def _diag_correction():
        row_a = jax.lax.broadcasted_iota(jnp.int32, (bi, half), 0)
        col_a = jax.lax.broadcasted_iota(jnp.int32, (bi, half), 1)
        diag = (jnp.sum(jnp.where(row_a == col_a, d_a, 0.0))
                + jnp.sum(jnp.where(row_a == col_a + half, d_b, 0.0)))
        out_ref[...] -= diag


def kernel(features, target):
    n, c = features.shape
    blk = _BLK if n % _BLK == 0 else n
    ni = n // blk
    njj = ni // 2 + 1

    fb = features.astype(jnp.bfloat16)
    tf = target.astype(jnp.float32)
    trow = tf.reshape(ni, 1, blk)
    tcol = jnp.broadcast_to(tf[:, None], (n, 128)).reshape(ni, blk, 128)

    sq_row, sq_col, fbm2 = pl.pallas_call(
        _norms_block,
        grid=(ni,),
        in_specs=[pl.BlockSpec((blk, c), lambda i: (i, 0))],
        out_specs=[
            pl.BlockSpec((1, 1, blk), lambda i: (i, 0, 0)),
            pl.BlockSpec((1, blk, 128), lambda i: (i, 0, 0)),
            pl.BlockSpec((blk, c), lambda i: (i, 0)),
        ],
        out_shape=[
            jax.ShapeDtypeStruct((ni, 1, blk), jnp.float32),
            jax.ShapeDtypeStruct((ni, blk, 128), jnp.float32),
            jax.ShapeDtypeStruct((n, c), jnp.bfloat16),
        ],
        compiler_params=pltpu.CompilerParams(
            dimension_semantics=("parallel",),
        ),
    )(fb)

    grid = (ni, njj)
    partials = pl.pallas_call(
        functools.partial(_loss_block, ni=ni),
        grid=grid,
        in_specs=[
            pl.BlockSpec((blk, c), lambda i, jj: (i, 0)),
            pl.BlockSpec((blk, c), lambda i, jj: ((i + jj) % ni, 0)),
            pl.BlockSpec((1, blk, 128), lambda i, jj: (i, 0, 0)),
            pl.BlockSpec((1, 1, blk), lambda i, jj: ((i + jj) % ni, 0, 0)),
            pl.BlockSpec((1, blk, 128), lambda i, jj: (i, 0, 0)),
            pl.BlockSpec((1, 1, blk), lambda i, jj: ((i + jj) % ni, 0, 0)),
        ],
        out_specs=pl.BlockSpec((1, 1, 128), lambda i, jj: (i, 0, 0)),
        out_shape=jax.ShapeDtypeStruct((ni, 1, 128), jnp.float32),
        compiler_params=pltpu.CompilerParams(
            dimension_semantics=("parallel", "arbitrary"),
        ),
    )(fb, fbm2, sq_col, sq_row, tcol, trow)

    t = n * (n - 1)
    return jnp.sum(partials[:, 0, 0]) / (2.0 * t)


# speedup vs baseline: 3.1365x; 1.0923x over previous
import functools

import jax
import jax.numpy as jnp
from jax.experimental import pallas as pl
from jax.experimental.pallas import tpu as pltpu

_MARGIN = 1.0
_BLK = 1024


def _norms_block(f_ref, fb_ref, fm2_ref, row_ref, col_ref):
    fb = f_ref[...].astype(jnp.bfloat16)
    fb_ref[...] = fb
    fm2_ref[...] = fb * jnp.bfloat16(-2.0)
    f32 = fb.astype(jnp.float32)
    sq2 = f32 * f32
    col = jnp.sum(sq2, axis=1, keepdims=True)
    ones_row = jnp.ones((1, f32.shape[1]), jnp.float32)
    row = jax.lax.dot_general(ones_row, sq2, (((1,), (1,)), ((), ())),
                              preferred_element_type=jnp.float32)
    row_ref[0] = row
    col_ref[0] = jnp.broadcast_to(col, col_ref.shape[1:])


def _loss_block(fi_ref, fj_ref, sqc_ref, sqr_ref, tcol_ref, trow_ref,
                out_ref, *, ni):
    jj = pl.program_id(1)

    @pl.when(jj == 0)
    def _init():
        out_ref[...] = jnp.zeros_like(out_ref)

    fi = fi_ref[...]
    bi = fi_ref.shape[0]
    bj = fj_ref.shape[0]
    half = bj // 2
    sqc = sqc_ref[0, :, 0:1]
    ti = tcol_ref[0, :, 0:1]

    def _half(lo):
        fj = fj_ref[pl.ds(lo, half), :]
        g2 = jax.lax.dot_general(fi, fj, (((1,), (1,)), ((), ())),
                                 preferred_element_type=jnp.float32)
        sqr = sqr_ref[0, :, pl.ds(lo, half)]
        d = jnp.maximum((sqc + sqr) + g2, 1e-20)
        same = ti == trow_ref[0, :, pl.ds(lo, half)]
        s = d * jax.lax.rsqrt(d)
        r = jnp.maximum(_MARGIN - s, 0.0)
        val = jnp.where(same, d, r * r)
        return d, jnp.sum(val)

    d_a, sum_a = _half(0)
    d_b, sum_b = _half(half)

    w = jnp.where((jj == 0) | (jj * 2 == ni), 1.0, 2.0)
    out_ref[...] += w * (sum_a + sum_b)

    @pl.when(jj == 0)
    def _diag_correction():
        row_a = jax.lax.broadcasted_iota(jnp.int32, (bi, half), 0)
        col_a = jax.lax.broadcasted_iota(jnp.int32, (bi, half), 1)
        diag = (jnp.sum(jnp.where(row_a == col_a, d_a, 0.0))
                + jnp.sum(jnp.where(row_a == col_a + half, d_b, 0.0)))
        out_ref[...] -= diag


def kernel(features, target):
    n, c = features.shape
    blk = _BLK if n % _BLK == 0 else n
    ni = n // blk
    njj = ni // 2 + 1

    tf = target.astype(jnp.float32)
    trow = tf.reshape(ni, 1, blk)
    tcol = jnp.broadcast_to(tf[:, None], (n, 128)).reshape(ni, blk, 128)

    fb, fbm2, sq_row, sq_col = pl.pallas_call(
        _norms_block,
        grid=(ni,),
        in_specs=[pl.BlockSpec((blk, c), lambda i: (i, 0))],
        out_specs=[
            pl.BlockSpec((blk, c), lambda i: (i, 0)),
            pl.BlockSpec((blk, c), lambda i: (i, 0)),
            pl.BlockSpec((1, 1, blk), lambda i: (i, 0, 0)),
            pl.BlockSpec((1, blk, 128), lambda i: (i, 0, 0)),
        ],
        out_shape=[
            jax.ShapeDtypeStruct((n, c), jnp.bfloat16),
            jax.ShapeDtypeStruct((n, c), jnp.bfloat16),
            jax.ShapeDtypeStruct((ni, 1, blk), jnp.float32),
            jax.ShapeDtypeStruct((ni, blk, 128), jnp.float32),
        ],
        compiler_params=pltpu.CompilerParams(
            dimension_semantics=("arbitrary",),
        ),
    )(features)

    grid = (ni, njj)
    partials = pl.pallas_call(
        functools.partial(_loss_block, ni=ni),
        grid=grid,
        in_specs=[
            pl.BlockSpec((blk, c), lambda i, jj: (i, 0)),
            pl.BlockSpec((blk, c), lambda i, jj: ((i + jj) % ni, 0)),
            pl.BlockSpec((1, blk, 128), lambda i, jj: (i, 0, 0)),
            pl.BlockSpec((1, 1, blk), lambda i, jj: ((i + jj) % ni, 0, 0)),
            pl.BlockSpec((1, blk, 128), lambda i, jj: (i, 0, 0)),
            pl.BlockSpec((1, 1, blk), lambda i, jj: ((i + jj) % ni, 0, 0)),
        ],
        out_specs=pl.BlockSpec((1, 1, 128), lambda i, jj: (i, 0, 0)),
        out_shape=jax.ShapeDtypeStruct((ni, 1, 128), jnp.float32),
        compiler_params=pltpu.CompilerParams(
            dimension_semantics=("arbitrary", "arbitrary"),
        ),
    )(fb, fbm2, sq_col, sq_row, tcol, trow)

    t = n * (n - 1)
    return jnp.sum(partials[:, 0, 0]) / (2.0 * t)


# speedup vs baseline: 3.2366x; 1.0319x over previous
import functools

import jax
import jax.numpy as jnp
from jax.experimental import pallas as pl
from jax.experimental.pallas import tpu as pltpu

_MARGIN = 1.0
_BLK = 1024
_NCHUNK = 4


def _norms_block(f_ref, fb_ref, fm2_ref, row_ref, col_ref):
    fb = f_ref[...].astype(jnp.bfloat16)
    fb_ref[...] = fb
    fm2_ref[...] = fb * jnp.bfloat16(-2.0)
    f32 = fb.astype(jnp.float32)
    sq2 = f32 * f32
    col = jnp.sum(sq2, axis=1, keepdims=True)
    ones_row = jnp.ones((1, f32.shape[1]), jnp.float32)
    row = jax.lax.dot_general(ones_row, sq2, (((1,), (1,)), ((), ())),
                              preferred_element_type=jnp.float32)
    row_ref[0] = row
    col_ref[0] = jnp.broadcast_to(col, col_ref.shape[1:])


def _loss_block(fi_ref, fj_ref, sqc_ref, sqr_ref, tcol_ref, trow_ref,
                out_ref, acc_ref, *, ni, njj):
    jj = pl.program_id(1)

    fi = fi_ref[...]
    bi = fi_ref.shape[0]
    bj = fj_ref.shape[0]
    chunk = bj // _NCHUNK
    sqc = sqc_ref[0, :, 0:1]
    ti = tcol_ref[0, :, 0:1]

    def _vrow(x):
        return jnp.sum(x.reshape(bi // 8, 8, chunk), axis=0)

    def _chunk(lo):
        fj = fj_ref[pl.ds(lo, chunk), :]
        g2 = jax.lax.dot_general(fi, fj, (((1,), (1,)), ((), ())),
                                 preferred_element_type=jnp.float32)
        sqr = sqr_ref[0, :, pl.ds(lo, chunk)]
        d = jnp.maximum((sqc + sqr) + g2, 1e-20)
        same = ti == trow_ref[0, :, pl.ds(lo, chunk)]
        s = d * jax.lax.rsqrt(d)
        r = jnp.maximum(_MARGIN - s, 0.0)
        val = jnp.where(same, d, r * r)
        return d, _vrow(val)

    los = list(range(0, bj, chunk))
    parts = [_chunk(lo) for lo in los]

    w = jnp.where((jj == 0) | (jj * 2 == ni), 1.0, 2.0)
    step = w * functools.reduce(lambda a, b: a + b, [p[1] for p in parts])

    @pl.when(jj == 0)
    def _diag_correction():
        row_a = jax.lax.broadcasted_iota(jnp.int32, (bi, chunk), 0)
        col_a = jax.lax.broadcasted_iota(jnp.int32, (bi, chunk), 1)
        diag = functools.reduce(
            lambda a, b: a + b,
            [_vrow(jnp.where(row_a == col_a + lo, p[0], 0.0))
             for lo, p in zip(los, parts)])
        acc_ref[...] = step - diag

    @pl.when(jj != 0)
    def _accumulate():
        acc_ref[...] += step

    @pl.when(jj == njj - 1)
    def _finalize():
        out_ref[...] = jnp.full(out_ref.shape, jnp.sum(acc_ref[...]),
                                dtype=out_ref.dtype)


def kernel(features, target):
    n, c = features.shape
    blk = _BLK if n % _BLK == 0 else n
    ni = n // blk
    njj = ni // 2 + 1

    tf = target.astype(jnp.float32)
    trow = tf.reshape(ni, 1, blk)
    tcol = jnp.broadcast_to(tf[:, None], (n, 128)).reshape(ni, blk, 128)

    fb, fbm2, sq_row, sq_col = pl.pallas_call(
        _norms_block,
        grid=(ni,),
        in_specs=[pl.BlockSpec((blk, c), lambda i: (i, 0))],
        out_specs=[
            pl.BlockSpec((blk, c), lambda i: (i, 0)),
            pl.BlockSpec((blk, c), lambda i: (i, 0)),
            pl.BlockSpec((1, 1, blk), lambda i: (i, 0, 0)),
            pl.BlockSpec((1, blk, 128), lambda i: (i, 0, 0)),
        ],
        out_shape=[
            jax.ShapeDtypeStruct((n, c), jnp.bfloat16),
            jax.ShapeDtypeStruct((n, c), jnp.bfloat16),
            jax.ShapeDtypeStruct((ni, 1, blk), jnp.float32),
            jax.ShapeDtypeStruct((ni, blk, 128), jnp.float32),
        ],
        compiler_params=pltpu.CompilerParams(
            dimension_semantics=("arbitrary",),
        ),
    )(features)

    grid = (ni, njj)
    partials = pl.pallas_call(
        functools.partial(_loss_block, ni=ni, njj=njj),
        grid=grid,
        in_specs=[
            pl.BlockSpec((blk, c), lambda i, jj: (i, 0)),
            pl.BlockSpec((blk, c), lambda i, jj: ((i + jj) % ni, 0)),
            pl.BlockSpec((1, blk, 128), lambda i, jj: (i, 0, 0)),
            pl.BlockSpec((1, 1, blk), lambda i, jj: ((i + jj) % ni, 0, 0)),
            pl.BlockSpec((1, blk, 128), lambda i, jj: (i, 0, 0)),
            pl.BlockSpec((1, 1, blk), lambda i, jj: ((i + jj) % ni, 0, 0)),
        ],
        out_specs=pl.BlockSpec((1, 1, 128), lambda i, jj: (i, 0, 0)),
        out_shape=jax.ShapeDtypeStruct((ni, 1, 128), jnp.float32),
        scratch_shapes=[pltpu.VMEM((8, blk // _NCHUNK), jnp.float32)],
        compiler_params=pltpu.CompilerParams(
            dimension_semantics=("arbitrary", "arbitrary"),
        ),
    )(fb, fbm2, sq_col, sq_row, tcol, trow)

    t = n * (n - 1)
    return jnp.sum(partials[:, 0, 0]) / (2.0 * t)
